# Initial kernel scaffold; baseline (speedup 1.0000x reference)
import jax
import jax.numpy as jnp
from jax.experimental import pallas as pl


def kernel(z_e, dictionary):
    raise NotImplementedError("write your pallas kernel here")


# speedup vs baseline: 29.2791x; 29.2791x over previous
import jax
import jax.numpy as jnp
from jax.experimental import pallas as pl
from jax.experimental.pallas import tpu as pltpu

_N = 256
_C = 64
_K = 4
_N_BINS = 33
_COEF_MAX = 2.0
_COMMIT = 0.25
_EPS = 1e-10


def _omp_body(ze_ref, dict_ref, zq_ref, tok_ref, loss_ref, dnb_scr, dntb_scr, g_scr):
    i = pl.program_id(0)
    blk = ze_ref.shape[2]

    @pl.when(i == 0)
    def _init():
        d = dict_ref[...]
        nrm = jnp.maximum(jnp.sqrt(jnp.sum(d * d, axis=0, keepdims=True)), _EPS)
        dn = d / nrm
        dnb = dn.astype(jnp.bfloat16)
        dnb_scr[...] = dnb
        dntb_scr[...] = dnb.T
        g_scr[...] = jnp.dot(dnb.T, dnb, preferred_element_type=jnp.float32)

    dnb = dnb_scr[...]
    dntb = dntb_scr[...]
    g = g_scr[...]
    sigt = ze_ref[0]

    h_bart = jnp.dot(dntb, sigt.astype(jnp.bfloat16),
                     preferred_element_type=jnp.float32)

    iota0 = jax.lax.broadcasted_iota(jnp.int32, (_N, blk), 0)

    ht = h_bart
    maskf = jnp.zeros((_N, blk), jnp.float32)
    L = {(0, 0): jnp.ones((1, blk), jnp.float32)}
    idxs, onehots, rows, hbs = [], [], [], []
    x = None
    for k in range(_K):
        score = jnp.abs(ht) * (1.0 - maskf)
        mx = jnp.max(score, axis=0, keepdims=True)
        idx = jnp.min(jnp.where(score == mx, iota0, _N), axis=0, keepdims=True)
        oh = (iota0 == idx).astype(jnp.float32)
        maskf = jnp.maximum(maskf, oh)
        idxs.append(idx)
        onehots.append(oh)
        hbs.append(jnp.sum(h_bart * oh, axis=0, keepdims=True))

        if k > 0:
            w = []
            for j in range(k):
                acc = jnp.sum(oh * rows[j], axis=0, keepdims=True)
                for m in range(j):
                    acc = acc - L[(j, m)] * w[m]
                if j > 0:
                    acc = acc / L[(j, j)]
                w.append(acc)
            for j in range(k):
                L[(k, j)] = w[j]
            ss = w[0] * w[0]
            for j in range(1, k):
                ss = ss + w[j] * w[j]
            L[(k, k)] = jnp.sqrt(jnp.maximum(1.0 - ss, 1e-12))

        y = []
        for j in range(k + 1):
            acc = hbs[j]
            for m in range(j):
                acc = acc - L[(j, m)] * y[m]
            if j > 0:
                acc = acc / L[(j, j)]
            y.append(acc)
        x = [None] * (k + 1)
        for j in range(k, -1, -1):
            acc = y[j]
            for m in range(j + 1, k + 1):
                acc = acc - L[(m, j)] * x[m]
            if j > 0:
                acc = acc / L[(j, j)]
            x[j] = acc

        if k < _K - 1:
            rows.append(jnp.dot(g, oh, preferred_element_type=jnp.float32, precision=jax.lax.Precision.HIGHEST))
            if k == 0:
                beta = x[0] * rows[0]
            else:
                def _b(a):
                    return a.astype(jnp.bfloat16).astype(jnp.float32)
                beta = _b(x[0]) * _b(rows[0])
                for j in range(1, k + 1):
                    beta = beta + _b(x[j]) * _b(rows[j])
            ht = h_bart - beta

    wc = None
    toks = []
    for k in range(_K):
        c = jnp.clip(x[k], -_COEF_MAX, _COEF_MAX)
        scaled = (c + _COEF_MAX) / (2.0 * _COEF_MAX)
        bini = jnp.clip(jnp.round(scaled * (_N_BINS - 1)).astype(jnp.int32),
                        0, _N_BINS - 1)
        step = 2.0 * _COEF_MAX / (_N_BINS - 1)
        cq = bini.astype(jnp.float32) * step - _COEF_MAX
        toks.append(idxs[k] * _N_BINS + bini)
        contrib = cq * onehots[k]
        wc = contrib if wc is None else wc + contrib

    zqt = jnp.dot(dnb, wc.astype(jnp.bfloat16),
                  preferred_element_type=jnp.float32)

    zq_ref[0] = sigt + (zqt - sigt)
    tok_ref[0] = jnp.concatenate(toks, axis=0)

    diff = zqt - sigt
    sq = jnp.sum(diff * diff, axis=(0, 1), keepdims=True)

    @pl.when(i == 0)
    def _first():
        loss_ref[...] = sq

    @pl.when(i > 0)
    def _rest():
        loss_ref[...] = loss_ref[...] + sq


def kernel(z_e, dictionary):
    B, C, H, W = z_e.shape
    HW = H * W
    ze3 = z_e.reshape(B, C, HW)

    zq3, tok3, losssum = pl.pallas_call(
        _omp_body,
        grid=(B,),
        in_specs=[
            pl.BlockSpec((1, C, HW), lambda i: (i, 0, 0)),
            pl.BlockSpec((C, _N), lambda i: (0, 0)),
        ],
        out_specs=[
            pl.BlockSpec((1, C, HW), lambda i: (i, 0, 0)),
            pl.BlockSpec((1, _K, HW), lambda i: (i, 0, 0)),
            pl.BlockSpec((1, 1), lambda i: (0, 0)),
        ],
        out_shape=[
            jax.ShapeDtypeStruct((B, C, HW), jnp.float32),
            jax.ShapeDtypeStruct((B, _K, HW), jnp.int32),
            jax.ShapeDtypeStruct((1, 1), jnp.float32),
        ],
        scratch_shapes=[
            pltpu.VMEM((C, _N), jnp.bfloat16),
            pltpu.VMEM((_N, C), jnp.bfloat16),
            pltpu.VMEM((_N, _N), jnp.float32),
        ],
    )(ze3, dictionary)

    z_q_ste = zq3.reshape(B, C, H, W)
    tokens = tok3.transpose(0, 2, 1).reshape(B, H, W, _K)
    m = losssum[0, 0] / (B * C * H * W)
    loss = m + _COMMIT * m
    return z_q_ste, loss, tokens


# speedup vs baseline: 29.8327x; 1.0189x over previous
import jax
import jax.numpy as jnp
from jax.experimental import pallas as pl
from jax.experimental.pallas import tpu as pltpu

_N = 256
_C = 64
_K = 4
_N_BINS = 33
_COEF_MAX = 2.0
_COMMIT = 0.25
_EPS = 1e-10


def _omp_body(ze_ref, dict_ref, zq_ref, tok_ref, loss_ref, dnb_scr, dntb_scr, g_scr):
    i = pl.program_id(0)
    blk = ze_ref.shape[2]

    @pl.when(i == 0)
    def _init():
        d = dict_ref[...]
        nrm = jnp.maximum(jnp.sqrt(jnp.sum(d * d, axis=0, keepdims=True)), _EPS)
        dn = d / nrm
        dnb = dn.astype(jnp.bfloat16)
        dnb_scr[...] = dnb
        dntb_scr[...] = dnb.T
        g_scr[...] = jnp.dot(dnb.T, dnb, preferred_element_type=jnp.float32)

    dnb = dnb_scr[...]
    dntb = dntb_scr[...]
    g = g_scr[...]
    sigt = ze_ref[0]

    h_bart = jnp.dot(dntb, sigt.astype(jnp.bfloat16),
                     preferred_element_type=jnp.float32)

    iota0 = jax.lax.broadcasted_iota(jnp.int32, (_N, blk), 0)

    ht = h_bart
    invf = jnp.ones((_N, blk), jnp.float32)
    L = {(0, 0): jnp.ones((1, blk), jnp.float32)}
    idxs, onehots, rows, rowsb, hbs = [], [], [], [], []
    x = None
    for k in range(_K):
        score = jnp.abs(ht) * invf
        mx = jnp.max(score, axis=0, keepdims=True)
        idx = jnp.min(jnp.where(score == mx, iota0, _N), axis=0, keepdims=True)
        oh = (iota0 == idx).astype(jnp.float32)
        invf = invf - oh
        idxs.append(idx)
        onehots.append(oh)
        hbs.append(jnp.sum(h_bart * oh, axis=0, keepdims=True))

        if k > 0:
            w = []
            for j in range(k):
                acc = jnp.sum(oh * rows[j], axis=0, keepdims=True)
                for m in range(j):
                    acc = acc - L[(j, m)] * w[m]
                if j > 0:
                    acc = acc / L[(j, j)]
                w.append(acc)
            for j in range(k):
                L[(k, j)] = w[j]
            ss = w[0] * w[0]
            for j in range(1, k):
                ss = ss + w[j] * w[j]
            L[(k, k)] = jnp.sqrt(jnp.maximum(1.0 - ss, 1e-12))

        y = []
        for j in range(k + 1):
            acc = hbs[j]
            for m in range(j):
                acc = acc - L[(j, m)] * y[m]
            if j > 0:
                acc = acc / L[(j, j)]
            y.append(acc)
        x = [None] * (k + 1)
        for j in range(k, -1, -1):
            acc = y[j]
            for m in range(j + 1, k + 1):
                acc = acc - L[(m, j)] * x[m]
            if j > 0:
                acc = acc / L[(j, j)]
            x[j] = acc

        if k < _K - 1:
            rows.append(jnp.dot(g, oh, preferred_element_type=jnp.float32, precision=jax.lax.Precision.HIGHEST))
            rowsb.append(rows[-1].astype(jnp.bfloat16).astype(jnp.float32))

            def _b(a):
                return a.astype(jnp.bfloat16).astype(jnp.float32)
            if k == 0:
                beta = x[0] * rows[0]
            else:
                beta = _b(x[0]) * rowsb[0]
                for j in range(1, k + 1):
                    beta = beta + _b(x[j]) * rowsb[j]
            ht = h_bart - beta

    wc = None
    toks = []
    for k in range(_K):
        c = jnp.clip(x[k], -_COEF_MAX, _COEF_MAX)
        scaled = (c + _COEF_MAX) / (2.0 * _COEF_MAX)
        bini = jnp.clip(jnp.round(scaled * (_N_BINS - 1)).astype(jnp.int32),
                        0, _N_BINS - 1)
        step = 2.0 * _COEF_MAX / (_N_BINS - 1)
        cq = bini.astype(jnp.float32) * step - _COEF_MAX
        toks.append(idxs[k] * _N_BINS + bini)
        contrib = cq * onehots[k]
        wc = contrib if wc is None else wc + contrib

    zqt = jnp.dot(dnb, wc.astype(jnp.bfloat16),
                  preferred_element_type=jnp.float32)

    zq_ref[0] = sigt + (zqt - sigt)
    tok_ref[0] = jnp.concatenate(toks, axis=0)

    diff = zqt - sigt
    sq = jnp.sum(diff * diff, axis=(0, 1), keepdims=True)

    @pl.when(i == 0)
    def _first():
        loss_ref[...] = sq

    @pl.when(i > 0)
    def _rest():
        loss_ref[...] = loss_ref[...] + sq


def kernel(z_e, dictionary):
    B, C, H, W = z_e.shape
    HW = H * W
    ze3 = z_e.reshape(B, C, HW)

    zq3, tok3, losssum = pl.pallas_call(
        _omp_body,
        grid=(B,),
        in_specs=[
            pl.BlockSpec((1, C, HW), lambda i: (i, 0, 0)),
            pl.BlockSpec((C, _N), lambda i: (0, 0)),
        ],
        out_specs=[
            pl.BlockSpec((1, C, HW), lambda i: (i, 0, 0)),
            pl.BlockSpec((1, _K, HW), lambda i: (i, 0, 0)),
            pl.BlockSpec((1, 1), lambda i: (0, 0)),
        ],
        out_shape=[
            jax.ShapeDtypeStruct((B, C, HW), jnp.float32),
            jax.ShapeDtypeStruct((B, _K, HW), jnp.int32),
            jax.ShapeDtypeStruct((1, 1), jnp.float32),
        ],
        scratch_shapes=[
            pltpu.VMEM((C, _N), jnp.bfloat16),
            pltpu.VMEM((_N, C), jnp.bfloat16),
            pltpu.VMEM((_N, _N), jnp.float32),
        ],
    )(ze3, dictionary)

    z_q_ste = zq3.reshape(B, C, H, W)
    tokens = tok3.transpose(0, 2, 1).reshape(B, H, W, _K)
    m = losssum[0, 0] / (B * C * H * W)
    loss = m + _COMMIT * m
    return z_q_ste, loss, tokens
